# baseline (speedup 1.0000x reference)
import functools

import jax
import jax.numpy as jnp
from jax import lax
from jax.experimental import pallas as pl
from jax.experimental.pallas import tpu as pltpu
from jax.experimental.pallas import tpu_sc as plsc

N = 10000
E = 320000
D = 128
H = 4
PH = D // H
MSG = 32
T = 4
FF = 512
SMALL = 1e-7
ZW = T * D
AW = 160

NC = 2
NS = 16
NW = NC * NS
EPW = E // NW
CH = 80
NCHUNK = EPW // CH
NZCHUNK = N // CH
DROWS = 1000
NDCHUNK = N // DROWS

_NBLK = 5
_BLK = N // _NBLK


def _z_kernel(x_ref, w_ref, b_ref, z_ref):
    z_ref[...] = jnp.maximum(
        jnp.dot(x_ref[...], w_ref[...], preferred_element_type=jnp.float32)
        + b_ref[...], 0.0).astype(jnp.bfloat16)


def _compute_z(x, wbd, bvec):
    return pl.pallas_call(
        _z_kernel,
        grid=(_NBLK,),
        in_specs=[
            pl.BlockSpec((_BLK, D), lambda i: (i, 0)),
            pl.BlockSpec((D, ZW), lambda i: (0, 0)),
            pl.BlockSpec((1, ZW), lambda i: (0, 0)),
        ],
        out_specs=pl.BlockSpec((_BLK, ZW), lambda i: (i, 0)),
        out_shape=jax.ShapeDtypeStruct((N, ZW), jnp.bfloat16),
    )(x, wbd, bvec)


def _sc_edge_phase(z, src, dst2, attr):
    mesh = plsc.VectorSubcoreMesh(core_axis_name="c", subcore_axis_name="s")

    @functools.partial(
        pl.kernel,
        mesh=mesh,
        out_type=jax.ShapeDtypeStruct((NC, N, AW), jnp.bfloat16),
        compiler_params=pltpu.CompilerParams(use_tc_tiling_on_sc=False,
                                             needs_layout_passes=False),
        scratch_types=[
            pltpu.VMEM_SHARED((N, AW), jnp.bfloat16),
            pltpu.VMEM((EPW,), jnp.int32),
            pltpu.VMEM((NCHUNK, CH), jnp.int32),
            pltpu.VMEM((T, CH), jnp.float32),
            pltpu.VMEM((T, CH), jnp.float32),
            pltpu.VMEM((CH, ZW), jnp.bfloat16),
            pltpu.VMEM((CH, ZW), jnp.bfloat16),
            pltpu.VMEM((CH, AW), jnp.bfloat16),
            pltpu.VMEM((CH, AW), jnp.bfloat16),
            pltpu.SemaphoreType.DMA,
            pltpu.SemaphoreType.DMA,
            pltpu.SemaphoreType.DMA,
            pltpu.SemaphoreType.DMA,
            pltpu.SemaphoreType.DMA,
            pltpu.SemaphoreType.DMA,
            pltpu.SemaphoreType.DMA,
            pltpu.SemaphoreType.DMA,
        ],
    )
    def edge_kernel(z_hbm, src_hbm, dst_hbm, attr_hbm, out_hbm,
                    acc, srcv, dstv, attr0, attr1, rows0, rows1, msg0, msg1,
                    s_src, s_dst, sa0, sa1, sg0, sg1, ss0, ss1):
        cid = lax.axis_index("c")
        sid = lax.axis_index("s")
        wid = sid * NC + cid

        zeros32 = jnp.zeros((32,), jnp.bfloat16)
        deg_one = plsc.bitcast(
            jnp.where(lax.iota(jnp.int32, 16) == 0,
                      jnp.float32(1.0), jnp.float32(0.0)), jnp.bfloat16)

        h_src = pltpu.async_copy(src_hbm.at[pl.ds(wid * EPW, EPW)], srcv,
                                 s_src)
        h_dst = pltpu.async_copy(dst_hbm.at[wid], dstv, s_dst)
        base = wid * EPW
        pltpu.async_copy(attr_hbm.at[:, pl.ds(base, CH)], attr0, sa0)
        pltpu.async_copy(attr_hbm.at[:, pl.ds(base + CH, CH)], attr1, sa1)

        @pl.loop(0, CH)
        def _(i):
            for j in range(AW // 32):
                msg0[i, pl.ds(j * 32, 32)] = zeros32

        @pl.loop(0, (NZCHUNK + NS - 1) // NS)
        def _(k):
            c = sid + k * NS

            @pl.when(c < NZCHUNK)
            def _():
                pltpu.sync_copy(msg0, acc.at[pl.ds(c * CH, CH)])

        plsc.subcore_barrier()
        h_src.wait()
        h_dst.wait()

        pltpu.async_copy(z_hbm.at[srcv.at[pl.ds(0, CH)]], rows0, sg0)
        pltpu.async_copy(z_hbm.at[srcv.at[pl.ds(CH, CH)]], rows1, sg1)

        def do_chunk(i, rows, msg, attrv, sg, sa, ss, reissue):
            pltpu.make_async_copy(z_hbm.at[srcv.at[pl.ds(0, CH)]],
                                  rows, sg).wait()
            pltpu.make_async_copy(attr_hbm.at[:, pl.ds(0, CH)],
                                  attrv, sa).wait()

            @pl.when(i >= 2)
            def _():
                pltpu.make_async_copy(msg, acc.at[dstv.at[0]], ss).wait()

            def bcast16(s):
                v = lax.broadcast(s, (16,))
                return plsc.pack(v, v, format=plsc.PackFormat.INTERLEAVED)

            @plsc.parallel_loop(0, CH // 16, unroll=2)
            def _(g):
                e0 = g * 16
                av0 = attrv[0, pl.ds(e0, 16)]
                av1 = attrv[1, pl.ds(e0, 16)]
                av2 = attrv[2, pl.ds(e0, 16)]
                av3 = attrv[3, pl.ds(e0, 16)]
                for j in range(16):
                    e = e0 + j
                    a0 = bcast16(av0[j])
                    a1 = bcast16(av1[j])
                    a2 = bcast16(av2[j])
                    a3 = bcast16(av3[j])
                    for blk in range(D // 32):
                        m = (a0 * rows[e, pl.ds(blk * 32, 32)]
                             + a1 * rows[e, pl.ds(D + blk * 32, 32)]
                             + a2 * rows[e, pl.ds(2 * D + blk * 32, 32)]
                             + a3 * rows[e, pl.ds(3 * D + blk * 32, 32)])
                        msg[e, pl.ds(blk * 32, 32)] = m
                    msg[e, pl.ds(D, 32)] = deg_one

            pltpu.async_copy(msg, acc.at[dstv.at[i]], ss, add=True)

            if reissue:
                @pl.when(i + 2 < NCHUNK)
                def _():
                    pltpu.async_copy(
                        z_hbm.at[srcv.at[pl.ds((i + 2) * CH, CH)]], rows, sg)
                    pltpu.async_copy(
                        attr_hbm.at[:, pl.ds(base + (i + 2) * CH, CH)],
                        attrv, sa)

        @pl.loop(0, NCHUNK // 2)
        def _(k):
            do_chunk(2 * k, rows0, msg0, attr0, sg0, sa0, ss0, True)
            do_chunk(2 * k + 1, rows1, msg1, attr1, sg1, sa1, ss1, True)

        if NCHUNK % 2:
            do_chunk(NCHUNK - 1, rows0, msg0, attr0, sg0, sa0, ss0, False)

        pltpu.make_async_copy(msg0, acc.at[dstv.at[0]], ss0).wait()
        pltpu.make_async_copy(msg1, acc.at[dstv.at[0]], ss1).wait()

        plsc.subcore_barrier()

        @pl.when(sid < NDCHUNK)
        def _():
            pltpu.sync_copy(acc.at[pl.ds(sid * DROWS, DROWS)],
                            out_hbm.at[cid, pl.ds(sid * DROWS, DROWS)])

    return edge_kernel(z, src, dst2, attr)


def _delta_kernel(degp_ref, delta_ref):
    deg = (degp_ref[0, :, D].astype(jnp.float32)
           + degp_ref[0, :, D + 1].astype(jnp.float32)
           + degp_ref[1, :, D].astype(jnp.float32)
           + degp_ref[1, :, D + 1].astype(jnp.float32))
    delta_ref[0, 0] = jnp.mean(jnp.log(deg + 1.0))


def _compute_delta(aggp):
    return pl.pallas_call(
        _delta_kernel,
        in_specs=[pl.BlockSpec((NC, N, AW), lambda: (0, 0, 0))],
        out_specs=pl.BlockSpec(memory_space=pltpu.SMEM),
        out_shape=jax.ShapeDtypeStruct((1, 1), jnp.float32),
    )(aggp)


def _tail_kernel(x_ref, aggp_ref, delta_ref, wid_ref, wamp_ref, watt_ref,
                 bo_ref, g_ref, bb_ref, al_ref, w1_ref, b1_ref, w2_ref,
                 b2_ref, o_ref):
    agg = (aggp_ref[0].astype(jnp.float32) + aggp_ref[1].astype(jnp.float32))
    agg128 = agg[:, :D]
    deg = agg[:, D:D + 1] + agg[:, D + 1:D + 2]
    ld = jnp.log(deg + 1.0)
    delta = delta_ref[0, 0]
    amp = ld / (delta + SMALL)
    att = delta / (ld + SMALL)
    new = (jnp.dot(agg128, wid_ref[...], preferred_element_type=jnp.float32)
           + amp * jnp.dot(agg128, wamp_ref[...],
                           preferred_element_type=jnp.float32)
           + att * jnp.dot(agg128, watt_ref[...],
                           preferred_element_type=jnp.float32)
           + bo_ref[...])
    alpha = al_ref[0, 0]
    node = x_ref[...] + alpha * new
    mu = jnp.mean(node, axis=1, keepdims=True)
    var = jnp.mean((node - mu) ** 2, axis=1, keepdims=True)
    nrm = (node - mu) * lax.rsqrt(var + 1e-5) * g_ref[...] + bb_ref[...]
    h1 = jax.nn.gelu(jnp.dot(nrm, w1_ref[...],
                             preferred_element_type=jnp.float32) + b1_ref[...])
    boom = jnp.dot(h1, w2_ref[...],
                   preferred_element_type=jnp.float32) + b2_ref[...]
    o_ref[...] = node + alpha * boom


def _dense_tail(x, aggp, delta, wid, wamp, watt, b_out, ln_g, ln_b, alpha,
                w1, b1, w2, b2):
    return pl.pallas_call(
        _tail_kernel,
        grid=(_NBLK,),
        in_specs=[
            pl.BlockSpec((_BLK, D), lambda i: (i, 0)),
            pl.BlockSpec((NC, _BLK, AW), lambda i: (0, i, 0)),
            pl.BlockSpec((1, 1), lambda i: (0, 0), memory_space=pltpu.SMEM),
            pl.BlockSpec((D, D), lambda i: (0, 0)),
            pl.BlockSpec((D, D), lambda i: (0, 0)),
            pl.BlockSpec((D, D), lambda i: (0, 0)),
            pl.BlockSpec((1, D), lambda i: (0, 0)),
            pl.BlockSpec((1, D), lambda i: (0, 0)),
            pl.BlockSpec((1, D), lambda i: (0, 0)),
            pl.BlockSpec((1, 1), lambda i: (0, 0), memory_space=pltpu.SMEM),
            pl.BlockSpec((D, FF), lambda i: (0, 0)),
            pl.BlockSpec((1, FF), lambda i: (0, 0)),
            pl.BlockSpec((FF, D), lambda i: (0, 0)),
            pl.BlockSpec((1, D), lambda i: (0, 0)),
        ],
        out_specs=pl.BlockSpec((_BLK, D), lambda i: (i, 0)),
        out_shape=jax.ShapeDtypeStruct((N, D), jnp.float32),
    )(x, aggp, delta, wid, wamp, watt, b_out.reshape(1, D), ln_g.reshape(1, D),
      ln_b.reshape(1, D), alpha, w1, b1.reshape(1, FF), w2, b2.reshape(1, D))


def kernel(x, edge_index, edge_attr, W_msg, b_msg, W_out, b_out,
           ln_g, ln_b, alpha, W1, b1, W2, b2):
    eye = jnp.eye(H, dtype=jnp.float32)
    wbd = jnp.einsum('htpm,hk->hptkm', W_msg, eye).reshape(D, ZW)
    bvec = jnp.transpose(b_msg, (1, 0, 2)).reshape(1, ZW)

    w3 = W_out.reshape(H, 3, MSG, D)
    wid = w3[:, 0].reshape(D, D)
    wamp = w3[:, 1].reshape(D, D)
    watt = w3[:, 2].reshape(D, D)

    src = edge_index[0]
    dst2 = edge_index[1].reshape(NW, NCHUNK, CH)
    attr = edge_attr.T

    z = _compute_z(x, wbd, bvec)
    aggp = _sc_edge_phase(z, src, dst2, attr)
    delta = _compute_delta(aggp)
    return _dense_tail(x, aggp, delta, wid, wamp, watt, b_out, ln_g, ln_b,
                       alpha.reshape(1, 1), W1, b1, W2, b2)


# speedup vs baseline: 1.0768x; 1.0768x over previous
import functools

import jax
import jax.numpy as jnp
from jax import lax
from jax.experimental import pallas as pl
from jax.experimental.pallas import tpu as pltpu
from jax.experimental.pallas import tpu_sc as plsc

N = 10000
E = 320000
D = 128
H = 4
PH = D // H
MSG = 32
T = 4
FF = 512
SMALL = 1e-7
ZW = T * D
AW = 160

NC = 2
NS = 16
NW = NC * NS
EPW = E // NW
CH = 80
NCHUNK = EPW // CH
NZCHUNK = N // CH
DROWS = 1000
NDCHUNK = N // DROWS

_NBLK = 5
_BLK = N // _NBLK


def _z_kernel(x_ref, w_ref, b_ref, z_ref):
    z_ref[...] = jnp.maximum(
        jnp.dot(x_ref[...], w_ref[...], preferred_element_type=jnp.float32)
        + b_ref[...], 0.0).astype(jnp.bfloat16)


def _compute_z(x, wbd, bvec):
    return pl.pallas_call(
        _z_kernel,
        grid=(_NBLK,),
        in_specs=[
            pl.BlockSpec((_BLK, D), lambda i: (i, 0)),
            pl.BlockSpec((D, ZW), lambda i: (0, 0)),
            pl.BlockSpec((1, ZW), lambda i: (0, 0)),
        ],
        out_specs=pl.BlockSpec((_BLK, ZW), lambda i: (i, 0)),
        out_shape=jax.ShapeDtypeStruct((N, ZW), jnp.bfloat16),
    )(x, wbd, bvec)


def _sc_edge_phase(z, src, dst2, attr):
    mesh = plsc.VectorSubcoreMesh(core_axis_name="c", subcore_axis_name="s")

    @functools.partial(
        pl.kernel,
        mesh=mesh,
        out_type=jax.ShapeDtypeStruct((NC, N, AW), jnp.bfloat16),
        compiler_params=pltpu.CompilerParams(use_tc_tiling_on_sc=False,
                                             needs_layout_passes=False),
        scratch_types=[
            pltpu.VMEM_SHARED((N, AW), jnp.bfloat16),
            pltpu.VMEM((EPW,), jnp.int32),
            pltpu.VMEM((NCHUNK, CH), jnp.int32),
            pltpu.VMEM((T, CH), jnp.float32),
            pltpu.VMEM((T, CH), jnp.float32),
            pltpu.VMEM((CH, ZW), jnp.bfloat16),
            pltpu.VMEM((CH, ZW), jnp.bfloat16),
            pltpu.VMEM((CH, AW), jnp.bfloat16),
            pltpu.VMEM((CH, AW), jnp.bfloat16),
            pltpu.SemaphoreType.DMA,
            pltpu.SemaphoreType.DMA,
            pltpu.SemaphoreType.DMA,
            pltpu.SemaphoreType.DMA,
            pltpu.SemaphoreType.DMA,
            pltpu.SemaphoreType.DMA,
            pltpu.SemaphoreType.DMA,
            pltpu.SemaphoreType.DMA,
        ],
    )
    def edge_kernel(z_hbm, src_hbm, dst_hbm, attr_hbm, out_hbm,
                    acc, srcv, dstv, attr0, attr1, rows0, rows1, msg0, msg1,
                    s_src, s_dst, sa0, sa1, sg0, sg1, ss0, ss1):
        cid = lax.axis_index("c")
        sid = lax.axis_index("s")
        wid = sid * NC + cid

        zeros32 = jnp.zeros((32,), jnp.bfloat16)
        deg_one = plsc.bitcast(
            jnp.where(lax.iota(jnp.int32, 16) == 0,
                      jnp.float32(1.0), jnp.float32(0.0)), jnp.bfloat16)

        h_src = pltpu.async_copy(src_hbm.at[pl.ds(wid * EPW, EPW)], srcv,
                                 s_src)
        h_dst = pltpu.async_copy(dst_hbm.at[wid], dstv, s_dst)
        base = wid * EPW
        pltpu.async_copy(attr_hbm.at[:, pl.ds(base, CH)], attr0, sa0)
        pltpu.async_copy(attr_hbm.at[:, pl.ds(base + CH, CH)], attr1, sa1)

        @pl.loop(0, CH)
        def _(i):
            for j in range(AW // 32):
                msg0[i, pl.ds(j * 32, 32)] = zeros32

        @pl.loop(0, (NZCHUNK + NS - 1) // NS)
        def _(k):
            c = sid + k * NS

            @pl.when(c < NZCHUNK)
            def _():
                pltpu.sync_copy(msg0, acc.at[pl.ds(c * CH, CH)])

        plsc.subcore_barrier()
        h_src.wait()
        h_dst.wait()

        pltpu.async_copy(z_hbm.at[srcv.at[pl.ds(0, CH)]], rows0, sg0)
        pltpu.async_copy(z_hbm.at[srcv.at[pl.ds(CH, CH)]], rows1, sg1)

        def do_chunk(i, rows, msg, attrv, sg, sa, ss, reissue):
            pltpu.make_async_copy(z_hbm.at[srcv.at[pl.ds(0, CH)]],
                                  rows, sg).wait()
            pltpu.make_async_copy(attr_hbm.at[:, pl.ds(0, CH)],
                                  attrv, sa).wait()

            @pl.when(i >= 2)
            def _():
                pltpu.make_async_copy(msg, acc.at[dstv.at[0]], ss).wait()

            def bcast16(s):
                v = lax.broadcast(s, (16,))
                return plsc.pack(v, v, format=plsc.PackFormat.INTERLEAVED)

            @plsc.parallel_loop(0, CH // 16)
            def _(g):
                e0 = g * 16
                av0 = attrv[0, pl.ds(e0, 16)]
                av1 = attrv[1, pl.ds(e0, 16)]
                av2 = attrv[2, pl.ds(e0, 16)]
                av3 = attrv[3, pl.ds(e0, 16)]
                for j in range(16):
                    e = e0 + j
                    a0 = bcast16(av0[j])
                    a1 = bcast16(av1[j])
                    a2 = bcast16(av2[j])
                    a3 = bcast16(av3[j])
                    for blk in range(D // 32):
                        m = (a0 * rows[e, pl.ds(blk * 32, 32)]
                             + a1 * rows[e, pl.ds(D + blk * 32, 32)]
                             + a2 * rows[e, pl.ds(2 * D + blk * 32, 32)]
                             + a3 * rows[e, pl.ds(3 * D + blk * 32, 32)])
                        msg[e, pl.ds(blk * 32, 32)] = m
                    msg[e, pl.ds(D, 32)] = deg_one

            pltpu.async_copy(msg, acc.at[dstv.at[i]], ss, add=True)

            if reissue:
                @pl.when(i + 2 < NCHUNK)
                def _():
                    pltpu.async_copy(
                        z_hbm.at[srcv.at[pl.ds((i + 2) * CH, CH)]], rows, sg)
                    pltpu.async_copy(
                        attr_hbm.at[:, pl.ds(base + (i + 2) * CH, CH)],
                        attrv, sa)

        @pl.loop(0, NCHUNK // 2)
        def _(k):
            do_chunk(2 * k, rows0, msg0, attr0, sg0, sa0, ss0, True)
            do_chunk(2 * k + 1, rows1, msg1, attr1, sg1, sa1, ss1, True)

        if NCHUNK % 2:
            do_chunk(NCHUNK - 1, rows0, msg0, attr0, sg0, sa0, ss0, False)

        pltpu.make_async_copy(msg0, acc.at[dstv.at[0]], ss0).wait()
        pltpu.make_async_copy(msg1, acc.at[dstv.at[0]], ss1).wait()

        plsc.subcore_barrier()

        @pl.when(sid < NDCHUNK)
        def _():
            pltpu.sync_copy(acc.at[pl.ds(sid * DROWS, DROWS)],
                            out_hbm.at[cid, pl.ds(sid * DROWS, DROWS)])

    return edge_kernel(z, src, dst2, attr)


def _delta_kernel(degp_ref, delta_ref):
    deg = (degp_ref[0, :, D].astype(jnp.float32)
           + degp_ref[0, :, D + 1].astype(jnp.float32)
           + degp_ref[1, :, D].astype(jnp.float32)
           + degp_ref[1, :, D + 1].astype(jnp.float32))
    delta_ref[0, 0] = jnp.mean(jnp.log(deg + 1.0))


def _compute_delta(aggp):
    return pl.pallas_call(
        _delta_kernel,
        in_specs=[pl.BlockSpec((NC, N, AW), lambda: (0, 0, 0))],
        out_specs=pl.BlockSpec(memory_space=pltpu.SMEM),
        out_shape=jax.ShapeDtypeStruct((1, 1), jnp.float32),
    )(aggp)


def _tail_kernel(x_ref, aggp_ref, delta_ref, wid_ref, wamp_ref, watt_ref,
                 bo_ref, g_ref, bb_ref, al_ref, w1_ref, b1_ref, w2_ref,
                 b2_ref, o_ref):
    agg = (aggp_ref[0].astype(jnp.float32) + aggp_ref[1].astype(jnp.float32))
    agg128 = agg[:, :D]
    deg = agg[:, D:D + 1] + agg[:, D + 1:D + 2]
    ld = jnp.log(deg + 1.0)
    delta = delta_ref[0, 0]
    amp = ld / (delta + SMALL)
    att = delta / (ld + SMALL)
    new = (jnp.dot(agg128, wid_ref[...], preferred_element_type=jnp.float32)
           + amp * jnp.dot(agg128, wamp_ref[...],
                           preferred_element_type=jnp.float32)
           + att * jnp.dot(agg128, watt_ref[...],
                           preferred_element_type=jnp.float32)
           + bo_ref[...])
    alpha = al_ref[0, 0]
    node = x_ref[...] + alpha * new
    mu = jnp.mean(node, axis=1, keepdims=True)
    var = jnp.mean((node - mu) ** 2, axis=1, keepdims=True)
    nrm = (node - mu) * lax.rsqrt(var + 1e-5) * g_ref[...] + bb_ref[...]
    h1 = jax.nn.gelu(jnp.dot(nrm, w1_ref[...],
                             preferred_element_type=jnp.float32) + b1_ref[...])
    boom = jnp.dot(h1, w2_ref[...],
                   preferred_element_type=jnp.float32) + b2_ref[...]
    o_ref[...] = node + alpha * boom


def _dense_tail(x, aggp, delta, wid, wamp, watt, b_out, ln_g, ln_b, alpha,
                w1, b1, w2, b2):
    return pl.pallas_call(
        _tail_kernel,
        grid=(_NBLK,),
        in_specs=[
            pl.BlockSpec((_BLK, D), lambda i: (i, 0)),
            pl.BlockSpec((NC, _BLK, AW), lambda i: (0, i, 0)),
            pl.BlockSpec((1, 1), lambda i: (0, 0), memory_space=pltpu.SMEM),
            pl.BlockSpec((D, D), lambda i: (0, 0)),
            pl.BlockSpec((D, D), lambda i: (0, 0)),
            pl.BlockSpec((D, D), lambda i: (0, 0)),
            pl.BlockSpec((1, D), lambda i: (0, 0)),
            pl.BlockSpec((1, D), lambda i: (0, 0)),
            pl.BlockSpec((1, D), lambda i: (0, 0)),
            pl.BlockSpec((1, 1), lambda i: (0, 0), memory_space=pltpu.SMEM),
            pl.BlockSpec((D, FF), lambda i: (0, 0)),
            pl.BlockSpec((1, FF), lambda i: (0, 0)),
            pl.BlockSpec((FF, D), lambda i: (0, 0)),
            pl.BlockSpec((1, D), lambda i: (0, 0)),
        ],
        out_specs=pl.BlockSpec((_BLK, D), lambda i: (i, 0)),
        out_shape=jax.ShapeDtypeStruct((N, D), jnp.float32),
    )(x, aggp, delta, wid, wamp, watt, b_out.reshape(1, D), ln_g.reshape(1, D),
      ln_b.reshape(1, D), alpha, w1, b1.reshape(1, FF), w2, b2.reshape(1, D))


def kernel(x, edge_index, edge_attr, W_msg, b_msg, W_out, b_out,
           ln_g, ln_b, alpha, W1, b1, W2, b2):
    eye = jnp.eye(H, dtype=jnp.float32)
    wbd = jnp.einsum('htpm,hk->hptkm', W_msg, eye).reshape(D, ZW)
    bvec = jnp.transpose(b_msg, (1, 0, 2)).reshape(1, ZW)

    w3 = W_out.reshape(H, 3, MSG, D)
    wid = w3[:, 0].reshape(D, D)
    wamp = w3[:, 1].reshape(D, D)
    watt = w3[:, 2].reshape(D, D)

    src = edge_index[0]
    dst2 = edge_index[1].reshape(NW, NCHUNK, CH)
    attr = edge_attr.T

    z = _compute_z(x, wbd, bvec)
    aggp = _sc_edge_phase(z, src, dst2, attr)
    delta = _compute_delta(aggp)
    return _dense_tail(x, aggp, delta, wid, wamp, watt, b_out, ln_g, ln_b,
                       alpha.reshape(1, 1), W1, b1, W2, b2)
